# baseline (speedup 1.0000x reference)
import functools

import jax
import jax.numpy as jnp
from jax import lax
from jax.experimental import pallas as pl
from jax.experimental.pallas import tpu as pltpu
from jax.experimental.pallas import tpu_sc as plsc

NC = 2
NS = 16
NW = NC * NS
L = 16
PAD = 112
UNROLL = 8
NACC = 4


def _plan(n):
    per_w = n // NW
    assert per_w * NW == n
    chunk = min(32768, per_w // 2)
    while per_w % chunk or (per_w // chunk) % 2:
        chunk //= 2
    assert chunk % (L * UNROLL) == 0
    return per_w, chunk, per_w // chunk


TCB_ROWS = 256
TC_COLS = 8192
TC_SPLIT = 8


@functools.lru_cache(maxsize=None)
def _tc_minmax_kernel(nrows, nblk, interpret=False):
    base = nrows // TCB_ROWS - nblk

    def body(x_ref, o_ref):
        k = pl.program_id(0)

        @pl.when(k == 0)
        def _():
            o_ref[0, :] = jnp.full((TC_COLS,), jnp.inf, jnp.float32)
            o_ref[1, :] = jnp.full((TC_COLS,), -jnp.inf, jnp.float32)

        blk = x_ref[...]
        o_ref[0, :] = jnp.minimum(o_ref[0, :], jnp.min(blk, axis=0))
        o_ref[1, :] = jnp.maximum(o_ref[1, :], jnp.max(blk, axis=0))

    return pl.pallas_call(
        body,
        grid=(nblk,),
        in_specs=[pl.BlockSpec((TCB_ROWS, TC_COLS), lambda k: (base + k, 0))],
        out_specs=pl.BlockSpec((2, TC_COLS), lambda k: (0, 0)),
        out_shape=jax.ShapeDtypeStruct((2, TC_COLS), jnp.float32),
        interpret=interpret,
    )


@functools.lru_cache(maxsize=None)
def _minmax_kernel(n, ndata, interpret=False):
    per_w, chunk, nch = _plan(ndata)
    vec_iters = chunk // (L * UNROLL)
    mesh = plsc.VectorSubcoreMesh(core_axis_name="c", subcore_axis_name="s")

    def body(x_hbm, out_hbm, buf0, buf1, mm, sem0, sem1):
        wid = lax.axis_index("c") * NS + lax.axis_index("s")
        base = wid * per_w
        pltpu.make_async_copy(x_hbm.at[pl.ds(base, chunk)], buf0, sem0).start()
        pltpu.make_async_copy(
            x_hbm.at[pl.ds(base + chunk, chunk)], buf1, sem1).start()

        def process(buf, carry):
            def vb(i, c):
                ms, xs = list(c[:NACC]), list(c[NACC:])
                for u in range(UNROLL):
                    v = buf[pl.ds((i * UNROLL + u) * L, L)]
                    ms[u % NACC] = jnp.minimum(ms[u % NACC], v)
                    xs[u % NACC] = jnp.maximum(xs[u % NACC], v)
                return (*ms, *xs)
            return lax.fori_loop(0, vec_iters, vb, carry)

        def pair(g2, carry):
            for b, (buf, sem) in enumerate(((buf0, sem0), (buf1, sem1))):
                g = 2 * g2 + b
                pltpu.make_async_copy(
                    x_hbm.at[pl.ds(base + g * chunk, chunk)], buf, sem).wait()
                carry = process(buf, carry)
                pltpu.make_async_copy(
                    x_hbm.at[pl.ds(base + (g + 2) * chunk, chunk)], buf,
                    sem).start()
            return carry

        inf = jnp.full((L,), jnp.inf, jnp.float32)
        ninf = jnp.full((L,), -jnp.inf, jnp.float32)
        carry = (inf,) * NACC + (ninf,) * NACC
        carry = lax.fori_loop(0, nch // 2 - 1, pair, carry)
        for b, (buf, sem) in enumerate(((buf0, sem0), (buf1, sem1))):
            g = nch - 2 + b
            pltpu.make_async_copy(
                x_hbm.at[pl.ds(base + g * chunk, chunk)], buf, sem).wait()
            carry = process(buf, carry)

        mn, mx = carry[0], carry[NACC]
        for a in range(1, NACC):
            mn = jnp.minimum(mn, carry[a])
            mx = jnp.maximum(mx, carry[NACC + a])
        mm[0, :] = mn
        mm[1, :] = mx
        pltpu.sync_copy(mm, out_hbm.at[wid])

    return pl.kernel(
        body,
        out_type=jax.ShapeDtypeStruct((NW, 2, L), jnp.float32),
        mesh=mesh,
        scratch_types=[
            pltpu.VMEM((chunk,), jnp.float32),
            pltpu.VMEM((chunk,), jnp.float32),
            pltpu.VMEM((2, L), jnp.float32),
            pltpu.SemaphoreType.DMA,
            pltpu.SemaphoreType.DMA,
        ],
        compiler_params=pltpu.CompilerParams(needs_layout_passes=False),
        interpret=interpret,
    )


@functools.lru_cache(maxsize=None)
def _hist_kernel(n, interpret=False):
    per_w, chunk, nch = _plan(n)
    vec_iters = chunk // (L * UNROLL)
    mesh = plsc.VectorSubcoreMesh(core_axis_name="c", subcore_axis_name="s")

    def body(x_hbm, mm_hbm, pp_hbm, out_hbm, buf0, buf1,
             hists, mmv, ppv, part, sem0, sem1):
        wid = lax.axis_index("c") * NS + lax.axis_index("s")
        base = wid * per_w
        pltpu.make_async_copy(x_hbm.at[pl.ds(base, chunk)], buf0, sem0).start()
        pltpu.make_async_copy(
            x_hbm.at[pl.ds(base + chunk, chunk)], buf1, sem1).start()
        pltpu.sync_copy(mm_hbm, mmv)
        pltpu.sync_copy(pp_hbm, ppv)
        mn = mmv[0, 0, :]
        mx = mmv[0, 1, :]
        for r in range(1, NW + 1):
            mn = jnp.minimum(mn, mmv[r, 0, :])
            mx = jnp.maximum(mx, mmv[r, 1, :])
        data_lo = jnp.broadcast_to(jnp.min(mn), (L,))
        data_hi = jnp.broadcast_to(jnp.max(mx), (L,))
        minv = ppv[0, :]
        maxv = ppv[1, :]
        binsf = ppv[2, :]
        use_data = minv == maxv
        lo = jnp.where(use_data, data_lo, minv)
        hi = jnp.where(use_data, data_hi, maxv)
        span = hi - lo
        safe = jnp.where(span == 0.0, jnp.ones((L,), jnp.float32), span)
        scale = binsf / safe
        lane_base = (lax.iota(jnp.int32, L) * PAD).astype(jnp.float32)
        off = lo * scale - lane_base

        one = jnp.ones((L,), jnp.int32)
        zero = jnp.zeros((L,), jnp.int32)
        for h in hists:
            for j in range(L * PAD // L):
                h[pl.ds(j * L, L)] = zero

        nh = len(hists)

        def process(buf):
            @plsc.parallel_loop(0, chunk // (nh * L), unroll=4 * UNROLL // nh)
            def _(i):
                for u, h in enumerate(hists):
                    v = buf[pl.ds((nh * i + u) * L, L)]
                    t = v * scale - off
                    plsc.addupdate_scatter(h, [t.astype(jnp.int32)], one)

        def pair(g2, carry):
            for b, (buf, sem) in enumerate(((buf0, sem0), (buf1, sem1))):
                g = 2 * g2 + b
                pltpu.make_async_copy(
                    x_hbm.at[pl.ds(base + g * chunk, chunk)], buf, sem).wait()
                process(buf)
                pltpu.make_async_copy(
                    x_hbm.at[pl.ds(base + (g + 2) * chunk, chunk)], buf,
                    sem).start()
            return carry

        lax.fori_loop(0, nch // 2 - 1, pair, 0)
        for b, (buf, sem) in enumerate(((buf0, sem0), (buf1, sem1))):
            g = nch - 2 + b
            pltpu.make_async_copy(
                x_hbm.at[pl.ds(base + g * chunk, chunk)], buf, sem).wait()
            process(buf)

        for j in range(PAD // L):
            acc = hists[0][pl.ds(j * L, L)]
            for h in hists[1:]:
                acc = acc + h[pl.ds(j * L, L)]
            for r in range(1, L):
                for h in hists:
                    acc = acc + h[pl.ds(r * PAD + j * L, L)]
            part[pl.ds(j * L, L)] = acc
        pltpu.sync_copy(part, out_hbm.at[wid])

    def wrapped(x_hbm, mm_hbm, pp_hbm, out_hbm, buf0, buf1,
                ha, hb, mmv, ppv, part, sem0, sem1):
        body(x_hbm, mm_hbm, pp_hbm, out_hbm, buf0, buf1,
             (ha, hb), mmv, ppv, part, sem0, sem1)

    return pl.kernel(
        wrapped,
        out_type=jax.ShapeDtypeStruct((NW, PAD), jnp.int32),
        mesh=mesh,
        scratch_types=[
            pltpu.VMEM((chunk,), jnp.float32),
            pltpu.VMEM((chunk,), jnp.float32),
            pltpu.VMEM((L * PAD,), jnp.int32),
            pltpu.VMEM((L * PAD,), jnp.int32),
            pltpu.VMEM((NW + 1, 2, L), jnp.float32),
            pltpu.VMEM((3, L), jnp.float32),
            pltpu.VMEM((PAD,), jnp.int32),
            pltpu.SemaphoreType.DMA,
            pltpu.SemaphoreType.DMA,
        ],
        compiler_params=pltpu.CompilerParams(needs_layout_passes=False),
        interpret=interpret,
    )


def kernel(x, bins, min_val, max_val, interpret=False):
    n = x.shape[0]
    k_sc = n - TC_SPLIT * TCB_ROWS * TC_COLS
    mm_sc = _minmax_kernel(n, k_sc, interpret)(x)
    tc = _tc_minmax_kernel(n // TC_COLS, TC_SPLIT, interpret)(
        x.reshape(-1, TC_COLS))
    mm_tc = jnp.stack([
        jnp.broadcast_to(jnp.min(tc[0]), (L,)),
        jnp.broadcast_to(jnp.max(tc[1]), (L,)),
    ])[None]
    mm = jnp.concatenate([mm_sc, mm_tc], axis=0)
    pp = jnp.stack([
        jnp.full((L,), min_val, jnp.float32),
        jnp.full((L,), max_val, jnp.float32),
        jnp.full((L,), jnp.asarray(bins, jnp.float32)),
    ])
    parts = _hist_kernel(n, interpret)(x, mm, pp)
    hist = parts.sum(axis=0)
    hist = hist.at[99].add(hist[100])
    return hist[:100].astype(x.dtype)


# speedup vs baseline: 1.6423x; 1.6423x over previous
import functools

import jax
import jax.numpy as jnp
from jax import lax
from jax.experimental import pallas as pl
from jax.experimental.pallas import tpu as pltpu
from jax.experimental.pallas import tpu_sc as plsc

NC = 2
NS = 16
NW = NC * NS
L = 16
PAD = 112
UNROLL = 8
NACC = 4


def _plan(n):
    per_w = n // NW
    assert per_w * NW == n
    chunk = min(32768, per_w // 2)
    while per_w % chunk or (per_w // chunk) % 2:
        chunk //= 2
    assert chunk % (L * UNROLL) == 0
    return per_w, chunk, per_w // chunk


@functools.lru_cache(maxsize=None)
def _minmax_kernel(n, interpret=False):
    per_w, chunk, nch = _plan(n)
    vec_iters = chunk // (L * UNROLL)
    mesh = plsc.VectorSubcoreMesh(core_axis_name="c", subcore_axis_name="s")

    def body(x_hbm, out_hbm, buf0, buf1, mm, sem0, sem1):
        wid = lax.axis_index("c") * NS + lax.axis_index("s")
        base = wid * per_w
        pltpu.make_async_copy(x_hbm.at[pl.ds(base, chunk)], buf0, sem0).start()
        pltpu.make_async_copy(
            x_hbm.at[pl.ds(base + chunk, chunk)], buf1, sem1).start()

        def process(buf, carry):
            def vb(i, c):
                ms, xs = list(c[:NACC]), list(c[NACC:])
                for u in range(UNROLL):
                    v = buf[pl.ds((i * UNROLL + u) * L, L)]
                    ms[u % NACC] = jnp.minimum(ms[u % NACC], v)
                    xs[u % NACC] = jnp.maximum(xs[u % NACC], v)
                return (*ms, *xs)
            return lax.fori_loop(0, vec_iters, vb, carry)

        def pair(g2, carry):
            for b, (buf, sem) in enumerate(((buf0, sem0), (buf1, sem1))):
                g = 2 * g2 + b
                pltpu.make_async_copy(
                    x_hbm.at[pl.ds(base + g * chunk, chunk)], buf, sem).wait()
                carry = process(buf, carry)
                pltpu.make_async_copy(
                    x_hbm.at[pl.ds(base + (g + 2) * chunk, chunk)], buf,
                    sem).start()
            return carry

        inf = jnp.full((L,), jnp.inf, jnp.float32)
        ninf = jnp.full((L,), -jnp.inf, jnp.float32)
        carry = (inf,) * NACC + (ninf,) * NACC
        carry = lax.fori_loop(0, nch // 2 - 1, pair, carry)
        for b, (buf, sem) in enumerate(((buf0, sem0), (buf1, sem1))):
            g = nch - 2 + b
            pltpu.make_async_copy(
                x_hbm.at[pl.ds(base + g * chunk, chunk)], buf, sem).wait()
            carry = process(buf, carry)

        mn, mx = carry[0], carry[NACC]
        for a in range(1, NACC):
            mn = jnp.minimum(mn, carry[a])
            mx = jnp.maximum(mx, carry[NACC + a])
        mm[0, :] = mn
        mm[1, :] = mx
        pltpu.sync_copy(mm, out_hbm.at[wid])

    return pl.kernel(
        body,
        out_type=jax.ShapeDtypeStruct((NW, 2, L), jnp.float32),
        mesh=mesh,
        scratch_types=[
            pltpu.VMEM((chunk,), jnp.float32),
            pltpu.VMEM((chunk,), jnp.float32),
            pltpu.VMEM((2, L), jnp.float32),
            pltpu.SemaphoreType.DMA,
            pltpu.SemaphoreType.DMA,
        ],
        compiler_params=pltpu.CompilerParams(needs_layout_passes=False),
        interpret=interpret,
    )


@functools.lru_cache(maxsize=None)
def _hist_kernel(n, interpret=False):
    per_w, chunk, nch = _plan(n)
    vec_iters = chunk // (L * UNROLL)
    mesh = plsc.VectorSubcoreMesh(core_axis_name="c", subcore_axis_name="s")

    def body(x_hbm, mm_hbm, pp_hbm, out_hbm, buf0, buf1,
             hists, mmv, ppv, part, sem0, sem1):
        wid = lax.axis_index("c") * NS + lax.axis_index("s")
        base = wid * per_w
        pltpu.make_async_copy(x_hbm.at[pl.ds(base, chunk)], buf0, sem0).start()
        pltpu.make_async_copy(
            x_hbm.at[pl.ds(base + chunk, chunk)], buf1, sem1).start()
        pltpu.sync_copy(mm_hbm, mmv)
        pltpu.sync_copy(pp_hbm, ppv)
        mn = mmv[0, 0, :]
        mx = mmv[0, 1, :]
        for r in range(1, NW):
            mn = jnp.minimum(mn, mmv[r, 0, :])
            mx = jnp.maximum(mx, mmv[r, 1, :])
        data_lo = jnp.broadcast_to(jnp.min(mn), (L,))
        data_hi = jnp.broadcast_to(jnp.max(mx), (L,))
        minv = ppv[0, :]
        maxv = ppv[1, :]
        binsf = ppv[2, :]
        use_data = minv == maxv
        lo = jnp.where(use_data, data_lo, minv)
        hi = jnp.where(use_data, data_hi, maxv)
        span = hi - lo
        safe = jnp.where(span == 0.0, jnp.ones((L,), jnp.float32), span)
        scale = binsf / safe
        lane_base = (lax.iota(jnp.int32, L) * PAD).astype(jnp.float32)
        off = lo * scale - lane_base

        one = jnp.ones((L,), jnp.int32)
        zero = jnp.zeros((L,), jnp.int32)
        for h in hists:
            for j in range(L * PAD // L):
                h[pl.ds(j * L, L)] = zero

        nh = len(hists)

        def process(buf):
            @plsc.parallel_loop(0, chunk // (nh * L), unroll=4 * UNROLL // nh)
            def _(i):
                for u, h in enumerate(hists):
                    v = buf[pl.ds((nh * i + u) * L, L)]
                    t = v * scale - off
                    plsc.addupdate_scatter(h, [t.astype(jnp.int32)], one)

        def pair(g2, carry):
            for b, (buf, sem) in enumerate(((buf0, sem0), (buf1, sem1))):
                g = 2 * g2 + b
                pltpu.make_async_copy(
                    x_hbm.at[pl.ds(base + g * chunk, chunk)], buf, sem).wait()
                process(buf)
                pltpu.make_async_copy(
                    x_hbm.at[pl.ds(base + (g + 2) * chunk, chunk)], buf,
                    sem).start()
            return carry

        lax.fori_loop(0, nch // 2 - 1, pair, 0)
        for b, (buf, sem) in enumerate(((buf0, sem0), (buf1, sem1))):
            g = nch - 2 + b
            pltpu.make_async_copy(
                x_hbm.at[pl.ds(base + g * chunk, chunk)], buf, sem).wait()
            process(buf)

        for j in range(PAD // L):
            acc = hists[0][pl.ds(j * L, L)]
            for h in hists[1:]:
                acc = acc + h[pl.ds(j * L, L)]
            for r in range(1, L):
                for h in hists:
                    acc = acc + h[pl.ds(r * PAD + j * L, L)]
            part[pl.ds(j * L, L)] = acc
        pltpu.sync_copy(part, out_hbm.at[wid])

    def wrapped(x_hbm, mm_hbm, pp_hbm, out_hbm, buf0, buf1,
                ha, hb, mmv, ppv, part, sem0, sem1):
        body(x_hbm, mm_hbm, pp_hbm, out_hbm, buf0, buf1,
             (ha, hb), mmv, ppv, part, sem0, sem1)

    return pl.kernel(
        wrapped,
        out_type=jax.ShapeDtypeStruct((NW, PAD), jnp.int32),
        mesh=mesh,
        scratch_types=[
            pltpu.VMEM((chunk,), jnp.float32),
            pltpu.VMEM((chunk,), jnp.float32),
            pltpu.VMEM((L * PAD,), jnp.int32),
            pltpu.VMEM((L * PAD,), jnp.int32),
            pltpu.VMEM((NW, 2, L), jnp.float32),
            pltpu.VMEM((3, L), jnp.float32),
            pltpu.VMEM((PAD,), jnp.int32),
            pltpu.SemaphoreType.DMA,
            pltpu.SemaphoreType.DMA,
        ],
        compiler_params=pltpu.CompilerParams(needs_layout_passes=False),
        interpret=interpret,
    )


def kernel(x, bins, min_val, max_val, interpret=False):
    n = x.shape[0]
    mm = _minmax_kernel(n, interpret)(x)
    pp = jnp.stack([
        jnp.full((L,), min_val, jnp.float32),
        jnp.full((L,), max_val, jnp.float32),
        jnp.full((L,), jnp.asarray(bins, jnp.float32)),
    ])
    parts = _hist_kernel(n, interpret)(x, mm, pp)
    hist = parts.sum(axis=0)
    hist = hist.at[99].add(hist[100])
    return hist[:100].astype(x.dtype)
